# baseline (speedup 1.0000x reference)
import jax
import jax.numpy as jnp
from jax import lax
from jax.experimental import pallas as pl
from jax.experimental.pallas import tpu as pltpu
from jax.experimental.pallas import tpu_sc as plsc

N_NODES = 10000
N_EDGES = 320000
D_IN = 128
D_HID = 256
DQ = 64
NQ = 4

NC = 2
NS = 16
CHUNK = 96
NJ = 210
NJH = NJ // 5
E_PAD = NS * NJ * CHUNK
N_ACC = 10240
ZROWS = N_ACC // NS

ROW_BLK = 2000


def _embed_body(x_ref, we_ref, be_ref, wm_ref, h_ref, hm4_ref):
    h = jnp.maximum(
        jnp.dot(x_ref[...], we_ref[...], preferred_element_type=jnp.float32)
        + be_ref[...],
        0.0,
    )
    hm = jnp.dot(h, wm_ref[...], preferred_element_type=jnp.float32)
    h_ref[...] = h
    for q in range(NQ):
        hm4_ref[q] = hm[:, q * DQ:(q + 1) * DQ]


def _embed(x, W_embed, b_embed, W_msg):
    grid = (N_NODES // ROW_BLK,)
    return pl.pallas_call(
        _embed_body,
        grid=grid,
        in_specs=[
            pl.BlockSpec((ROW_BLK, D_IN), lambda i: (i, 0)),
            pl.BlockSpec((D_IN, D_HID), lambda i: (0, 0)),
            pl.BlockSpec((1, D_HID), lambda i: (0, 0)),
            pl.BlockSpec((D_HID, D_HID), lambda i: (0, 0)),
        ],
        out_specs=[
            pl.BlockSpec((ROW_BLK, D_HID), lambda i: (i, 0)),
            pl.BlockSpec((NQ, ROW_BLK, DQ), lambda i: (0, i, 0)),
        ],
        out_shape=[
            jax.ShapeDtypeStruct((N_NODES, D_HID), jnp.float32),
            jax.ShapeDtypeStruct((NQ, N_ACC, DQ), jnp.float32),
        ],
    )(x, W_embed, b_embed, W_msg)


NBUF = 6


def _agg_body(hm4, srcr, dstr, zr, agg4, src_v, dst_v, rows_v, table, acc,
              gs0, gs1, gs2, gs3, gs4, gs5, ss0, ss1, ss2, ss3, ss4, ss5):
    c = lax.axis_index("c")
    s = lax.axis_index("s")
    gsems = [gs0, gs1, gs2, gs3, gs4, gs5]
    ssems = [ss0, ss1, ss2, ss3, ss4, ss5]

    def gather(j, b):
        pltpu.async_copy(table.at[src_v.at[j]], rows_v.at[b], gsems[b])

    def drain_gather(j, b):
        pltpu.make_async_copy(
            table.at[src_v.at[j]], rows_v.at[b], gsems[b]).wait()

    def scatter(j, b):
        pltpu.async_copy(rows_v.at[b], acc.at[dst_v.at[j]], ssems[b], add=True)

    def drain_scatter(j, b):
        pltpu.make_async_copy(
            rows_v.at[b], acc.at[dst_v.at[j]], ssems[b]).wait()

    for qp in range(NQ // NC):
        q = NC * c + qp
        pltpu.sync_copy(hm4.at[q].at[pl.ds(s * ZROWS, ZROWS)],
                        table.at[pl.ds(s * ZROWS, ZROWS)])
        pltpu.sync_copy(zr, acc.at[pl.ds(s * ZROWS, ZROWS)])
        plsc.subcore_barrier()

        for p in range(NJ // NJH):
            pltpu.sync_copy(srcr.at[s].at[pl.ds(p * NJH, NJH)], src_v)
            pltpu.sync_copy(dstr.at[s].at[pl.ds(p * NJH, NJH)], dst_v)

            for b in range(NBUF - 1):
                gather(b, b)

            def body4(jj, carry):
                for b in range(NBUF):
                    j = NBUF * jj + b
                    drain_gather(j, b)
                    scatter(j, b)

                    @pl.when(j >= 1)
                    def _():
                        drain_scatter(j - 1, (b - 1) % NBUF)

                    @pl.when(j + NBUF - 1 <= NJH - 1)
                    def _():
                        gather(j + NBUF - 1, (b + NBUF - 1) % NBUF)
                return carry

            lax.fori_loop(0, NJH // NBUF, body4, 0)
            drain_scatter(NJH - 1, (NJH - 1) % NBUF)

        plsc.subcore_barrier()
        pltpu.sync_copy(acc.at[pl.ds(s * ZROWS, ZROWS)],
                        agg4.at[q].at[pl.ds(s * ZROWS, ZROWS)])
        plsc.subcore_barrier()


def _aggregate(hm4, src_r, dst_r, zeros_blk):
    mesh = plsc.VectorSubcoreMesh(
        core_axis_name="c", subcore_axis_name="s", num_cores=NC, num_subcores=NS)
    k = pl.kernel(
        _agg_body,
        out_type=jax.ShapeDtypeStruct((NQ, N_ACC, DQ), jnp.float32),
        mesh=mesh,
        scratch_types=[
            pltpu.VMEM((NJH, CHUNK), jnp.int32),
            pltpu.VMEM((NJH, CHUNK), jnp.int32),
            pltpu.VMEM((NBUF, CHUNK, DQ), jnp.float32),
            pltpu.VMEM_SHARED((N_ACC, DQ), jnp.float32),
            pltpu.VMEM_SHARED((N_ACC, DQ), jnp.float32),
        ] + [pltpu.SemaphoreType.DMA] * (2 * NBUF),
        compiler_params=pltpu.CompilerParams(use_tc_tiling_on_sc=False),
    )
    return k(hm4, src_r, dst_r, zeros_blk)


def _out_body(h_ref, agg4_ref, wo_ref, bo_ref, out_ref):
    acc = jnp.dot(h_ref[...], wo_ref[:D_HID], preferred_element_type=jnp.float32)
    for q in range(NQ):
        acc += jnp.dot(
            agg4_ref[q],
            wo_ref[D_HID + q * DQ:D_HID + (q + 1) * DQ],
            preferred_element_type=jnp.float32,
        )
    out_ref[...] = jnp.maximum(acc + bo_ref[...], 0.0)


def _output_mlp(h, agg4, W_out, b_out):
    grid = (N_NODES // ROW_BLK,)
    return pl.pallas_call(
        _out_body,
        grid=grid,
        in_specs=[
            pl.BlockSpec((ROW_BLK, D_HID), lambda i: (i, 0)),
            pl.BlockSpec((NQ, ROW_BLK, DQ), lambda i: (0, i, 0)),
            pl.BlockSpec((2 * D_HID, D_HID), lambda i: (0, 0)),
            pl.BlockSpec((1, D_HID), lambda i: (0, 0)),
        ],
        out_specs=pl.BlockSpec((ROW_BLK, D_HID), lambda i: (i, 0)),
        out_shape=jax.ShapeDtypeStruct((N_NODES, D_HID), jnp.float32),
    )(h, agg4, W_out, b_out)


@jax.jit
def kernel(x, edge_index, W_embed, b_embed, W_msg, W_out, b_out):
    src = edge_index[0]
    dst = edge_index[1]
    pad = E_PAD - N_EDGES
    src_r = jnp.concatenate([src, jnp.zeros((pad,), jnp.int32)]).reshape(
        NS, NJ, CHUNK)
    dst_r = jnp.concatenate(
        [dst, jnp.full((pad,), N_NODES, jnp.int32)]).reshape(NS, NJ, CHUNK)
    zeros_blk = jnp.zeros((ZROWS, DQ), jnp.float32)

    h, hm4 = _embed(x, W_embed, b_embed[None], W_msg)
    agg4 = _aggregate(hm4, src_r, dst_r, zeros_blk)
    return _output_mlp(h, agg4, W_out, b_out[None])


# speedup vs baseline: 1.0099x; 1.0099x over previous
import jax
import jax.numpy as jnp
from jax import lax
from jax.experimental import pallas as pl
from jax.experimental.pallas import tpu as pltpu
from jax.experimental.pallas import tpu_sc as plsc

N_NODES = 10000
N_EDGES = 320000
D_IN = 128
D_HID = 256
DQ = 64
NQ = 4

NC = 2
NS = 16
CHUNK = 112
NJ = 180
NJH = NJ // 4
E_PAD = NS * NJ * CHUNK
N_ACC = 10240
ZROWS = N_ACC // NS

ROW_BLK = 2000


def _embed_body(x_ref, we_ref, be_ref, wm_ref, h_ref, hm4_ref):
    h = jnp.maximum(
        jnp.dot(x_ref[...], we_ref[...], preferred_element_type=jnp.float32)
        + be_ref[...],
        0.0,
    )
    hm = jnp.dot(h, wm_ref[...], preferred_element_type=jnp.float32)
    h_ref[...] = h
    for q in range(NQ):
        hm4_ref[q] = hm[:, q * DQ:(q + 1) * DQ]


def _embed(x, W_embed, b_embed, W_msg):
    grid = (N_NODES // ROW_BLK,)
    return pl.pallas_call(
        _embed_body,
        grid=grid,
        in_specs=[
            pl.BlockSpec((ROW_BLK, D_IN), lambda i: (i, 0)),
            pl.BlockSpec((D_IN, D_HID), lambda i: (0, 0)),
            pl.BlockSpec((1, D_HID), lambda i: (0, 0)),
            pl.BlockSpec((D_HID, D_HID), lambda i: (0, 0)),
        ],
        out_specs=[
            pl.BlockSpec((ROW_BLK, D_HID), lambda i: (i, 0)),
            pl.BlockSpec((NQ, ROW_BLK, DQ), lambda i: (0, i, 0)),
        ],
        out_shape=[
            jax.ShapeDtypeStruct((N_NODES, D_HID), jnp.float32),
            jax.ShapeDtypeStruct((NQ, N_ACC, DQ), jnp.float32),
        ],
    )(x, W_embed, b_embed, W_msg)


NBUF = 5


def _agg_body(hm4, srcr, dstr, zr, agg4, src_v, dst_v, rows_v, table, acc,
              gs0, gs1, gs2, gs3, gs4, ss0, ss1, ss2, ss3, ss4):
    c = lax.axis_index("c")
    s = lax.axis_index("s")
    gsems = [gs0, gs1, gs2, gs3, gs4]
    ssems = [ss0, ss1, ss2, ss3, ss4]

    def gather(j, b):
        pltpu.async_copy(table.at[src_v.at[j]], rows_v.at[b], gsems[b])

    def drain_gather(j, b):
        pltpu.make_async_copy(
            table.at[src_v.at[j]], rows_v.at[b], gsems[b]).wait()

    def scatter(j, b):
        pltpu.async_copy(rows_v.at[b], acc.at[dst_v.at[j]], ssems[b], add=True)

    def drain_scatter(j, b):
        pltpu.make_async_copy(
            rows_v.at[b], acc.at[dst_v.at[j]], ssems[b]).wait()

    for qp in range(NQ // NC):
        q = NC * c + qp
        pltpu.sync_copy(hm4.at[q].at[pl.ds(s * ZROWS, ZROWS)],
                        table.at[pl.ds(s * ZROWS, ZROWS)])
        pltpu.sync_copy(zr, acc.at[pl.ds(s * ZROWS, ZROWS)])
        plsc.subcore_barrier()

        for p in range(NJ // NJH):
            pltpu.sync_copy(srcr.at[s].at[pl.ds(p * NJH, NJH)], src_v)
            pltpu.sync_copy(dstr.at[s].at[pl.ds(p * NJH, NJH)], dst_v)

            for b in range(NBUF - 1):
                gather(b, b)

            def body4(jj, carry):
                for b in range(NBUF):
                    j = NBUF * jj + b
                    drain_gather(j, b)
                    scatter(j, b)

                    @pl.when(j >= 1)
                    def _():
                        drain_scatter(j - 1, (b - 1) % NBUF)

                    @pl.when(j + NBUF - 1 <= NJH - 1)
                    def _():
                        gather(j + NBUF - 1, (b + NBUF - 1) % NBUF)
                return carry

            lax.fori_loop(0, NJH // NBUF, body4, 0)
            drain_scatter(NJH - 1, (NJH - 1) % NBUF)

        plsc.subcore_barrier()
        pltpu.sync_copy(acc.at[pl.ds(s * ZROWS, ZROWS)],
                        agg4.at[q].at[pl.ds(s * ZROWS, ZROWS)])
        plsc.subcore_barrier()


def _aggregate(hm4, src_r, dst_r, zeros_blk):
    mesh = plsc.VectorSubcoreMesh(
        core_axis_name="c", subcore_axis_name="s", num_cores=NC, num_subcores=NS)
    k = pl.kernel(
        _agg_body,
        out_type=jax.ShapeDtypeStruct((NQ, N_ACC, DQ), jnp.float32),
        mesh=mesh,
        scratch_types=[
            pltpu.VMEM((NJH, CHUNK), jnp.int32),
            pltpu.VMEM((NJH, CHUNK), jnp.int32),
            pltpu.VMEM((NBUF, CHUNK, DQ), jnp.float32),
            pltpu.VMEM_SHARED((N_ACC, DQ), jnp.float32),
            pltpu.VMEM_SHARED((N_ACC, DQ), jnp.float32),
        ] + [pltpu.SemaphoreType.DMA] * (2 * NBUF),
        compiler_params=pltpu.CompilerParams(use_tc_tiling_on_sc=False),
    )
    return k(hm4, src_r, dst_r, zeros_blk)


def _out_body(h_ref, agg4_ref, wo_ref, bo_ref, out_ref):
    acc = jnp.dot(h_ref[...], wo_ref[:D_HID], preferred_element_type=jnp.float32)
    for q in range(NQ):
        acc += jnp.dot(
            agg4_ref[q],
            wo_ref[D_HID + q * DQ:D_HID + (q + 1) * DQ],
            preferred_element_type=jnp.float32,
        )
    out_ref[...] = jnp.maximum(acc + bo_ref[...], 0.0)


def _output_mlp(h, agg4, W_out, b_out):
    grid = (N_NODES // ROW_BLK,)
    return pl.pallas_call(
        _out_body,
        grid=grid,
        in_specs=[
            pl.BlockSpec((ROW_BLK, D_HID), lambda i: (i, 0)),
            pl.BlockSpec((NQ, ROW_BLK, DQ), lambda i: (0, i, 0)),
            pl.BlockSpec((2 * D_HID, D_HID), lambda i: (0, 0)),
            pl.BlockSpec((1, D_HID), lambda i: (0, 0)),
        ],
        out_specs=pl.BlockSpec((ROW_BLK, D_HID), lambda i: (i, 0)),
        out_shape=jax.ShapeDtypeStruct((N_NODES, D_HID), jnp.float32),
    )(h, agg4, W_out, b_out)


@jax.jit
def kernel(x, edge_index, W_embed, b_embed, W_msg, W_out, b_out):
    src = edge_index[0]
    dst = edge_index[1]
    pad = E_PAD - N_EDGES
    src_r = jnp.concatenate([src, jnp.zeros((pad,), jnp.int32)]).reshape(
        NS, NJ, CHUNK)
    dst_r = jnp.concatenate(
        [dst, jnp.full((pad,), N_NODES, jnp.int32)]).reshape(NS, NJ, CHUNK)
    zeros_blk = jnp.zeros((ZROWS, DQ), jnp.float32)

    h, hm4 = _embed(x, W_embed, b_embed[None], W_msg)
    agg4 = _aggregate(hm4, src_r, dst_r, zeros_blk)
    return _output_mlp(h, agg4, W_out, b_out[None])
